# baseline (device time: 8037 ns/iter reference)
import jax
import jax.numpy as jnp
from jax import lax
from jax.experimental import pallas as pl
from jax.experimental.pallas import tpu as pltpu

N_DEV = 4
N_CHUNK = 8


def kernel(x):
    m, n = x.shape
    cm = m // N_CHUNK

    def body(x_hbm, out_hbm, x_vmem, out_vmem, halo_ref, in_sems, out_sems,
             send_sems, recv_sems):
        my_pos = lax.axis_index("i")
        has_left = my_pos > 0
        has_right = my_pos < N_DEV - 1
        left = my_pos - 1
        right = my_pos + 1

        def chunk_in(c):
            return pltpu.make_async_copy(
                x_hbm.at[pl.ds(c * cm, cm), :],
                x_vmem.at[pl.ds(c * cm, cm), :],
                in_sems.at[c],
            )

        def chunk_out(c):
            return pltpu.make_async_copy(
                out_vmem.at[pl.ds(c * cm, cm), :],
                out_hbm.at[pl.ds(c * cm, cm), :],
                out_sems.at[c],
            )

        for c in range(N_CHUNK):
            chunk_in(c).start()

        barrier_sem = pltpu.get_barrier_semaphore()
        left_tgt = jnp.maximum(left, 0)
        right_tgt = jnp.minimum(right, N_DEV - 1)
        pl.semaphore_signal(
            barrier_sem, inc=1,
            device_id=(left_tgt,), device_id_type=pl.DeviceIdType.MESH,
        )
        pl.semaphore_signal(
            barrier_sem, inc=1,
            device_id=(right_tgt,), device_id_type=pl.DeviceIdType.MESH,
        )
        pl.semaphore_wait(barrier_sem, 2)

        def mk_right():
            return pltpu.make_async_remote_copy(
                src_ref=x_hbm.at[pl.ds(m - 1, 1), :],
                dst_ref=halo_ref.at[0],
                send_sem=send_sems.at[0],
                recv_sem=recv_sems.at[0],
                device_id=(right_tgt,),
                device_id_type=pl.DeviceIdType.MESH,
            )

        def mk_left():
            return pltpu.make_async_remote_copy(
                src_ref=x_hbm.at[pl.ds(0, 1), :],
                dst_ref=halo_ref.at[1],
                send_sem=send_sems.at[1],
                recv_sem=recv_sems.at[1],
                device_id=(left_tgt,),
                device_id_type=pl.DeviceIdType.MESH,
            )

        @pl.when(has_right)
        def _():
            mk_right().start()

        @pl.when(has_left)
        def _():
            mk_left().start()

        chunk_in(0).wait()
        for c in range(N_CHUNK):
            if c < N_CHUNK - 1:
                chunk_in(c + 1).wait()
            lo = c * cm if c > 0 else 1
            hi = (c + 1) * cm if c < N_CHUNK - 1 else m - 1
            rows = hi - lo
            out_vmem[pl.ds(lo, rows), :] = (
                0.25 * (
                    x_vmem[pl.ds(lo - 1, rows), :]
                    + x_vmem[pl.ds(lo + 1, rows), :]
                )
                + 0.5 * x_vmem[pl.ds(lo, rows), :]
            ).astype(out_vmem.dtype)
            if 0 < c < N_CHUNK - 1:
                chunk_out(c).start()

        @pl.when(has_left)
        def _():
            mk_right().wait_recv()
            out_vmem[pl.ds(0, 1), :] = (
                0.25 * (halo_ref[0] + x_vmem[pl.ds(1, 1), :])
                + 0.5 * x_vmem[pl.ds(0, 1), :]
            ).astype(out_vmem.dtype)

        @pl.when(jnp.logical_not(has_left))
        def _():
            out_vmem[pl.ds(0, 1), :] = x_vmem[pl.ds(0, 1), :].astype(
                out_vmem.dtype
            )

        @pl.when(has_right)
        def _():
            mk_left().wait_recv()
            out_vmem[pl.ds(m - 1, 1), :] = (
                0.25 * (x_vmem[pl.ds(m - 2, 1), :] + halo_ref[1])
                + 0.5 * x_vmem[pl.ds(m - 1, 1), :]
            ).astype(out_vmem.dtype)

        @pl.when(jnp.logical_not(has_right))
        def _():
            out_vmem[pl.ds(m - 1, 1), :] = x_vmem[pl.ds(m - 1, 1), :].astype(
                out_vmem.dtype
            )

        chunk_out(0).start()
        chunk_out(N_CHUNK - 1).start()

        @pl.when(has_right)
        def _():
            mk_right().wait_send()

        @pl.when(has_left)
        def _():
            mk_left().wait_send()

        for c in range(N_CHUNK):
            chunk_out(c).wait()

    return pl.pallas_call(
        body,
        out_shape=jax.ShapeDtypeStruct((m, n), jnp.bfloat16),
        in_specs=[pl.BlockSpec(memory_space=pl.ANY)],
        out_specs=pl.BlockSpec(memory_space=pl.ANY),
        scratch_shapes=[
            pltpu.VMEM((m, n), x.dtype),
            pltpu.VMEM((m, n), jnp.bfloat16),
            pltpu.VMEM((2, 1, n), x.dtype),
            pltpu.SemaphoreType.DMA((N_CHUNK,)),
            pltpu.SemaphoreType.DMA((N_CHUNK,)),
            pltpu.SemaphoreType.DMA((2,)),
            pltpu.SemaphoreType.DMA((2,)),
        ],
        compiler_params=pltpu.CompilerParams(collective_id=0),
    )(x)


# device time: 7686 ns/iter; 1.0457x vs baseline; 1.0457x over previous
import jax
import jax.numpy as jnp
from jax import lax
from jax.experimental import pallas as pl
from jax.experimental.pallas import tpu as pltpu

N_DEV = 4
N_CHUNK = 4


def kernel(x):
    m, n = x.shape
    cm = m // N_CHUNK

    def body(x_hbm, out_hbm, x_vmem, out_vmem, halo_ref, in_sems, out_sems,
             send_sems, recv_sems):
        my_pos = lax.axis_index("i")
        has_left = my_pos > 0
        has_right = my_pos < N_DEV - 1
        left = my_pos - 1
        right = my_pos + 1

        def chunk_in(c):
            return pltpu.make_async_copy(
                x_hbm.at[pl.ds(c * cm, cm), :],
                x_vmem.at[pl.ds(c * cm, cm), :],
                in_sems.at[c],
            )

        def chunk_out(c):
            return pltpu.make_async_copy(
                out_vmem.at[pl.ds(c * cm, cm), :],
                out_hbm.at[pl.ds(c * cm, cm), :],
                out_sems.at[c],
            )

        chunk_in(N_CHUNK - 1).start()
        for c in range(N_CHUNK - 1):
            chunk_in(c).start()

        barrier_sem = pltpu.get_barrier_semaphore()
        left_tgt = jnp.maximum(left, 0)
        right_tgt = jnp.minimum(right, N_DEV - 1)
        pl.semaphore_signal(
            barrier_sem, inc=1,
            device_id=(left_tgt,), device_id_type=pl.DeviceIdType.MESH,
        )
        pl.semaphore_signal(
            barrier_sem, inc=1,
            device_id=(right_tgt,), device_id_type=pl.DeviceIdType.MESH,
        )
        pl.semaphore_wait(barrier_sem, 2)

        def mk_right():
            return pltpu.make_async_remote_copy(
                src_ref=x_vmem.at[pl.ds(m - 1, 1), :],
                dst_ref=halo_ref.at[0],
                send_sem=send_sems.at[0],
                recv_sem=recv_sems.at[0],
                device_id=(right_tgt,),
                device_id_type=pl.DeviceIdType.MESH,
            )

        def mk_left():
            return pltpu.make_async_remote_copy(
                src_ref=x_vmem.at[pl.ds(0, 1), :],
                dst_ref=halo_ref.at[1],
                send_sem=send_sems.at[1],
                recv_sem=recv_sems.at[1],
                device_id=(left_tgt,),
                device_id_type=pl.DeviceIdType.MESH,
            )

        chunk_in(N_CHUNK - 1).wait()

        @pl.when(has_right)
        def _():
            mk_right().start()

        chunk_in(0).wait()

        @pl.when(has_left)
        def _():
            mk_left().start()

        for c in range(N_CHUNK):
            if c < N_CHUNK - 2:
                chunk_in(c + 1).wait()
            lo = c * cm if c > 0 else 1
            hi = (c + 1) * cm if c < N_CHUNK - 1 else m - 1
            rows = hi - lo
            out_vmem[pl.ds(lo, rows), :] = (
                0.25 * (
                    x_vmem[pl.ds(lo - 1, rows), :]
                    + x_vmem[pl.ds(lo + 1, rows), :]
                )
                + 0.5 * x_vmem[pl.ds(lo, rows), :]
            ).astype(out_vmem.dtype)
            if 0 < c < N_CHUNK - 1:
                chunk_out(c).start()

        @pl.when(has_left)
        def _():
            mk_right().wait_recv()
            out_vmem[pl.ds(0, 1), :] = (
                0.25 * (halo_ref[0] + x_vmem[pl.ds(1, 1), :])
                + 0.5 * x_vmem[pl.ds(0, 1), :]
            ).astype(out_vmem.dtype)

        @pl.when(jnp.logical_not(has_left))
        def _():
            out_vmem[pl.ds(0, 1), :] = x_vmem[pl.ds(0, 1), :].astype(
                out_vmem.dtype
            )

        @pl.when(has_right)
        def _():
            mk_left().wait_recv()
            out_vmem[pl.ds(m - 1, 1), :] = (
                0.25 * (x_vmem[pl.ds(m - 2, 1), :] + halo_ref[1])
                + 0.5 * x_vmem[pl.ds(m - 1, 1), :]
            ).astype(out_vmem.dtype)

        @pl.when(jnp.logical_not(has_right))
        def _():
            out_vmem[pl.ds(m - 1, 1), :] = x_vmem[pl.ds(m - 1, 1), :].astype(
                out_vmem.dtype
            )

        chunk_out(0).start()
        chunk_out(N_CHUNK - 1).start()

        @pl.when(has_right)
        def _():
            mk_right().wait_send()

        @pl.when(has_left)
        def _():
            mk_left().wait_send()

        for c in range(N_CHUNK):
            chunk_out(c).wait()

    return pl.pallas_call(
        body,
        out_shape=jax.ShapeDtypeStruct((m, n), jnp.bfloat16),
        in_specs=[pl.BlockSpec(memory_space=pltpu.MemorySpace.HBM)],
        out_specs=pl.BlockSpec(memory_space=pltpu.MemorySpace.HBM),
        scratch_shapes=[
            pltpu.VMEM((m, n), x.dtype),
            pltpu.VMEM((m, n), jnp.bfloat16),
            pltpu.VMEM((2, 1, n), x.dtype),
            pltpu.SemaphoreType.DMA((N_CHUNK,)),
            pltpu.SemaphoreType.DMA((N_CHUNK,)),
            pltpu.SemaphoreType.DMA((2,)),
            pltpu.SemaphoreType.DMA((2,)),
        ],
        compiler_params=pltpu.CompilerParams(collective_id=0),
    )(x)


# device time: 7658 ns/iter; 1.0495x vs baseline; 1.0037x over previous
import jax
import jax.numpy as jnp
from jax import lax
from jax.experimental import pallas as pl
from jax.experimental.pallas import tpu as pltpu

N_DEV = 4
N_CHUNK = 4


def kernel(x):
    m, n = x.shape
    cm = m // N_CHUNK

    def body(x_hbm, out_hbm, x_vmem, out_vmem, halo_ref, vmem_hog, in_sems,
             out_sems, send_sems, recv_sems):
        my_pos = lax.axis_index("i")
        has_left = my_pos > 0
        has_right = my_pos < N_DEV - 1
        left = my_pos - 1
        right = my_pos + 1

        def chunk_in(c):
            return pltpu.make_async_copy(
                x_hbm.at[pl.ds(c * cm, cm), :],
                x_vmem.at[pl.ds(c * cm, cm), :],
                in_sems.at[c],
            )

        def chunk_out(c):
            return pltpu.make_async_copy(
                out_vmem.at[pl.ds(c * cm, cm), :],
                out_hbm.at[pl.ds(c * cm, cm), :],
                out_sems.at[c],
            )

        chunk_in(N_CHUNK - 1).start()
        for c in range(N_CHUNK - 1):
            chunk_in(c).start()

        barrier_sem = pltpu.get_barrier_semaphore()
        left_tgt = jnp.maximum(left, 0)
        right_tgt = jnp.minimum(right, N_DEV - 1)
        pl.semaphore_signal(
            barrier_sem, inc=1,
            device_id=(left_tgt,), device_id_type=pl.DeviceIdType.MESH,
        )
        pl.semaphore_signal(
            barrier_sem, inc=1,
            device_id=(right_tgt,), device_id_type=pl.DeviceIdType.MESH,
        )
        pl.semaphore_wait(barrier_sem, 2)

        def mk_right():
            return pltpu.make_async_remote_copy(
                src_ref=x_vmem.at[pl.ds(m - 1, 1), :],
                dst_ref=halo_ref.at[0],
                send_sem=send_sems.at[0],
                recv_sem=recv_sems.at[0],
                device_id=(right_tgt,),
                device_id_type=pl.DeviceIdType.MESH,
            )

        def mk_left():
            return pltpu.make_async_remote_copy(
                src_ref=x_vmem.at[pl.ds(0, 1), :],
                dst_ref=halo_ref.at[1],
                send_sem=send_sems.at[1],
                recv_sem=recv_sems.at[1],
                device_id=(left_tgt,),
                device_id_type=pl.DeviceIdType.MESH,
            )

        chunk_in(N_CHUNK - 1).wait()

        @pl.when(has_right)
        def _():
            mk_right().start()

        chunk_in(0).wait()

        @pl.when(has_left)
        def _():
            mk_left().start()

        for c in range(N_CHUNK):
            if c < N_CHUNK - 2:
                chunk_in(c + 1).wait()
            lo = c * cm if c > 0 else 1
            hi = (c + 1) * cm if c < N_CHUNK - 1 else m - 1
            rows = hi - lo
            out_vmem[pl.ds(lo, rows), :] = (
                0.25 * (
                    x_vmem[pl.ds(lo - 1, rows), :]
                    + x_vmem[pl.ds(lo + 1, rows), :]
                )
                + 0.5 * x_vmem[pl.ds(lo, rows), :]
            ).astype(out_vmem.dtype)
            if 0 < c < N_CHUNK - 1:
                chunk_out(c).start()

        @pl.when(has_left)
        def _():
            mk_right().wait_recv()
            out_vmem[pl.ds(0, 1), :] = (
                0.25 * (halo_ref[0] + x_vmem[pl.ds(1, 1), :])
                + 0.5 * x_vmem[pl.ds(0, 1), :]
            ).astype(out_vmem.dtype)

        @pl.when(jnp.logical_not(has_left))
        def _():
            out_vmem[pl.ds(0, 1), :] = x_vmem[pl.ds(0, 1), :].astype(
                out_vmem.dtype
            )

        @pl.when(has_right)
        def _():
            mk_left().wait_recv()
            out_vmem[pl.ds(m - 1, 1), :] = (
                0.25 * (x_vmem[pl.ds(m - 2, 1), :] + halo_ref[1])
                + 0.5 * x_vmem[pl.ds(m - 1, 1), :]
            ).astype(out_vmem.dtype)

        @pl.when(jnp.logical_not(has_right))
        def _():
            out_vmem[pl.ds(m - 1, 1), :] = x_vmem[pl.ds(m - 1, 1), :].astype(
                out_vmem.dtype
            )

        chunk_out(0).start()
        chunk_out(N_CHUNK - 1).start()

        @pl.when(has_right)
        def _():
            mk_right().wait_send()

        @pl.when(has_left)
        def _():
            mk_left().wait_send()

        for c in range(N_CHUNK):
            chunk_out(c).wait()

    return pl.pallas_call(
        body,
        out_shape=jax.ShapeDtypeStruct((m, n), jnp.bfloat16),
        in_specs=[pl.BlockSpec(memory_space=pltpu.MemorySpace.HBM)],
        out_specs=pl.BlockSpec(memory_space=pltpu.MemorySpace.HBM),
        scratch_shapes=[
            pltpu.VMEM((m, n), x.dtype),
            pltpu.VMEM((m, n), jnp.bfloat16),
            pltpu.VMEM((2, 1, n), x.dtype),
            pltpu.VMEM((14592, 512), jnp.float32),
            pltpu.SemaphoreType.DMA((N_CHUNK,)),
            pltpu.SemaphoreType.DMA((N_CHUNK,)),
            pltpu.SemaphoreType.DMA((2,)),
            pltpu.SemaphoreType.DMA((2,)),
        ],
        compiler_params=pltpu.CompilerParams(collective_id=0),
    )(x)


# device time: 7392 ns/iter; 1.0873x vs baseline; 1.0360x over previous
import jax
import jax.numpy as jnp
from jax import lax
from jax.experimental import pallas as pl
from jax.experimental.pallas import tpu as pltpu

N_DEV = 4


def kernel(x):
    m, n = x.shape

    def body(x_ref, out_ref, halo_ref, send_sems, recv_sems):
        my_pos = lax.axis_index("i")
        has_left = my_pos > 0
        has_right = my_pos < N_DEV - 1
        left = my_pos - 1
        right = my_pos + 1

        barrier_sem = pltpu.get_barrier_semaphore()
        left_tgt = jnp.maximum(left, 0)
        right_tgt = jnp.minimum(right, N_DEV - 1)
        pl.semaphore_signal(
            barrier_sem, inc=1,
            device_id=(left_tgt,), device_id_type=pl.DeviceIdType.MESH,
        )
        pl.semaphore_signal(
            barrier_sem, inc=1,
            device_id=(right_tgt,), device_id_type=pl.DeviceIdType.MESH,
        )
        pl.semaphore_wait(barrier_sem, 2)

        def mk_right():
            return pltpu.make_async_remote_copy(
                src_ref=x_ref.at[pl.ds(m - 1, 1), :],
                dst_ref=halo_ref.at[0],
                send_sem=send_sems.at[0],
                recv_sem=recv_sems.at[0],
                device_id=(right_tgt,),
                device_id_type=pl.DeviceIdType.MESH,
            )

        def mk_left():
            return pltpu.make_async_remote_copy(
                src_ref=x_ref.at[pl.ds(0, 1), :],
                dst_ref=halo_ref.at[1],
                send_sem=send_sems.at[1],
                recv_sem=recv_sems.at[1],
                device_id=(left_tgt,),
                device_id_type=pl.DeviceIdType.MESH,
            )

        @pl.when(has_right)
        def _():
            mk_right().start()

        @pl.when(has_left)
        def _():
            mk_left().start()

        out_ref[pl.ds(1, m - 2), :] = (
            0.25 * (x_ref[pl.ds(0, m - 2), :] + x_ref[pl.ds(2, m - 2), :])
            + 0.5 * x_ref[pl.ds(1, m - 2), :]
        ).astype(out_ref.dtype)

        @pl.when(has_left)
        def _():
            mk_right().wait_recv()
            out_ref[pl.ds(0, 1), :] = (
                0.25 * (halo_ref[0] + x_ref[pl.ds(1, 1), :])
                + 0.5 * x_ref[pl.ds(0, 1), :]
            ).astype(out_ref.dtype)

        @pl.when(jnp.logical_not(has_left))
        def _():
            out_ref[pl.ds(0, 1), :] = x_ref[pl.ds(0, 1), :].astype(out_ref.dtype)

        @pl.when(has_right)
        def _():
            mk_left().wait_recv()
            out_ref[pl.ds(m - 1, 1), :] = (
                0.25 * (x_ref[pl.ds(m - 2, 1), :] + halo_ref[1])
                + 0.5 * x_ref[pl.ds(m - 1, 1), :]
            ).astype(out_ref.dtype)

        @pl.when(jnp.logical_not(has_right))
        def _():
            out_ref[pl.ds(m - 1, 1), :] = x_ref[pl.ds(m - 1, 1), :].astype(
                out_ref.dtype
            )

        @pl.when(has_right)
        def _():
            mk_right().wait_send()

        @pl.when(has_left)
        def _():
            mk_left().wait_send()

    return pl.pallas_call(
        body,
        out_shape=jax.ShapeDtypeStruct((m, n), jnp.bfloat16),
        in_specs=[pl.BlockSpec(memory_space=pltpu.VMEM)],
        out_specs=pl.BlockSpec(memory_space=pltpu.VMEM),
        scratch_shapes=[
            pltpu.VMEM((2, 1, n), x.dtype),
            pltpu.SemaphoreType.DMA((2,)),
            pltpu.SemaphoreType.DMA((2,)),
        ],
        compiler_params=pltpu.CompilerParams(collective_id=0),
    )(x)
